# Initial kernel scaffold; baseline (speedup 1.0000x reference)
import jax
import jax.numpy as jnp
from jax.experimental import pallas as pl


def kernel(timestamps, train_timestamp, a, b, v, phi, h):
    raise NotImplementedError("write your pallas kernel here")


# speedup vs baseline: 2.8649x; 2.8649x over previous
import functools

import jax
import jax.numpy as jnp
from jax import lax
from jax.experimental import pallas as pl
from jax.experimental.pallas import tpu as pltpu
from jax.experimental.pallas import tpu_sc as plsc

NKEY = 32
NC = 2
NS = 16
L = 16
NW = NC * NS
CHUNK = 4096


def _sincos(x):
    y = x * 0.6366197723675814 + 0.5
    k = y.astype(jnp.int32)
    k = k - jnp.where(y < 0.0, 1, 0)
    fn = k.astype(jnp.float32)
    r = x - fn * 1.5707962513
    r = r - fn * 7.54979013e-8
    r2 = r * r
    s = -1.9515295891e-4
    s = s * r2 + 8.3321608736e-3
    s = s * r2 - 1.6666654611e-1
    s = s * r2 * r + r
    c = 2.44331571e-5
    c = c * r2 - 1.38873163e-3
    c = c * r2 + 4.16666457e-2
    c = c * r2 * r2 - 0.5 * r2 + 1.0
    swap = (k & 1) != 0
    sv = jnp.where(swap, c, s)
    cv = jnp.where(swap, s, c)
    sv = jnp.where((k & 2) != 0, -sv, sv)
    cv = jnp.where(((k + 1) & 2) != 0, -cv, cv)
    return sv, cv


def _sc_body(ts_hbm, tab_hbm, ao_hbm, bo_hbm, vo_hbm, po_hbm, ho_hbm,
             tab_v, ts_v, ao_v, bo_v, vo_v, po_v, ho_v, per_w):
    cid = lax.axis_index("c")
    sid = lax.axis_index("s")
    wid = sid * NC + cid
    base = wid * per_w
    pltpu.sync_copy(tab_hbm, tab_v)

    def chunk_body(ci, _):
        off = base + ci * CHUNK
        pltpu.sync_copy(ts_hbm.at[pl.ds(off, CHUNK)], ts_v)

        def vec_body(i, _):
            ts = ts_v[pl.ds(i * L, L)]
            idx = jnp.zeros((L,), jnp.int32)
            for step in (16, 8, 4, 2, 1):
                tv = plsc.load_gather(tab_v, [idx + step])
                idx = jnp.where(tv <= ts, idx + step, idx)
            tt = plsc.load_gather(tab_v, [idx])
            pa = plsc.load_gather(tab_v, [idx + 32])
            pb = plsc.load_gather(tab_v, [idx + 64])
            pv = plsc.load_gather(tab_v, [idx + 96])
            pp = plsc.load_gather(tab_v, [idx + 128])
            ph = plsc.load_gather(tab_v, [idx + 160])
            pacc = plsc.load_gather(tab_v, [idx + 192])
            pom = plsc.load_gather(tab_v, [idx + 224])
            psin = plsc.load_gather(tab_v, [idx + 256])
            pcos = plsc.load_gather(tab_v, [idx + 288])
            dt = ts - tt
            v_out = pv + pacc * dt
            phi_out = pp + pom * dt
            om = pom + 1e-6
            s, c = _sincos(phi_out)
            g = pv / om
            sl = pl.ds(i * L, L)
            ao_v[sl] = pa + g * (s - psin)
            bo_v[sl] = pb - g * (c - pcos)
            vo_v[sl] = v_out
            po_v[sl] = phi_out
            ho_v[sl] = ph
            return 0

        lax.fori_loop(0, CHUNK // L, vec_body, 0)
        dst = pl.ds(off, CHUNK)
        pltpu.sync_copy(ao_v, ao_hbm.at[dst])
        pltpu.sync_copy(bo_v, bo_hbm.at[dst])
        pltpu.sync_copy(vo_v, vo_hbm.at[dst])
        pltpu.sync_copy(po_v, po_hbm.at[dst])
        pltpu.sync_copy(ho_v, ho_hbm.at[dst])
        return 0

    lax.fori_loop(0, per_w // CHUNK, chunk_body, 0)


def kernel(timestamps, train_timestamp, a, b, v, phi, h):
    q = timestamps.shape[0]
    delta = jnp.diff(train_timestamp)
    acc = jnp.diff(v) / delta
    omega = jnp.diff(phi) / delta
    acc = jnp.concatenate([acc, acc[-1:]])
    omega = jnp.concatenate([omega, omega[-1:]])
    tab = jnp.concatenate(
        [train_timestamp, a, b, v, phi, h, acc, omega,
         jnp.sin(phi), jnp.cos(phi)]).astype(jnp.float32)

    grain = NW * CHUNK
    qp = ((q + grain - 1) // grain) * grain
    ts = timestamps
    if qp != q:
        ts = jnp.pad(ts, (0, qp - q))
    per_w = qp // NW

    mesh = plsc.VectorSubcoreMesh(core_axis_name="c", subcore_axis_name="s",
                                  num_cores=NC, num_subcores=NS)
    out = jax.ShapeDtypeStruct((qp,), jnp.float32)
    run = pl.kernel(
        functools.partial(_sc_body, per_w=per_w),
        out_type=(out, out, out, out, out),
        mesh=mesh,
        compiler_params=pltpu.CompilerParams(needs_layout_passes=False),
        scratch_types=[
            pltpu.VMEM((10 * NKEY,), jnp.float32),
            pltpu.VMEM((CHUNK,), jnp.float32),
            pltpu.VMEM((CHUNK,), jnp.float32),
            pltpu.VMEM((CHUNK,), jnp.float32),
            pltpu.VMEM((CHUNK,), jnp.float32),
            pltpu.VMEM((CHUNK,), jnp.float32),
            pltpu.VMEM((CHUNK,), jnp.float32),
        ],
    )
    a_out, b_out, v_out, phi_out, h_out = run(ts, tab)
    if qp != q:
        a_out, b_out, v_out, phi_out, h_out = (
            x[:q] for x in (a_out, b_out, v_out, phi_out, h_out))
    return (a_out, b_out, v_out, phi_out, h_out)


# speedup vs baseline: 7.9594x; 2.7782x over previous
import functools

import jax
import jax.numpy as jnp
from jax import lax
from jax.experimental import pallas as pl
from jax.experimental.pallas import tpu as pltpu
from jax.experimental.pallas import tpu_sc as plsc

NKEY = 32
NC = 2
NS = 16
L = 16
NW = NC * NS
CHUNK = 4096


def _sincos(x):
    y = x * 0.6366197723675814 + 0.5
    k = y.astype(jnp.int32)
    k = k - jnp.where(y < 0.0, 1, 0)
    fn = k.astype(jnp.float32)
    r = x - fn * 1.5707962513
    r = r - fn * 7.54979013e-8
    r2 = r * r
    s = -1.9515295891e-4
    s = s * r2 + 8.3321608736e-3
    s = s * r2 - 1.6666654611e-1
    s = s * r2 * r + r
    c = 2.44331571e-5
    c = c * r2 - 1.38873163e-3
    c = c * r2 + 4.16666457e-2
    c = c * r2 * r2 - 0.5 * r2 + 1.0
    swap = (k & 1) != 0
    sv = jnp.where(swap, c, s)
    cv = jnp.where(swap, s, c)
    sv = jnp.where((k & 2) != 0, -sv, sv)
    cv = jnp.where(((k + 1) & 2) != 0, -cv, cv)
    return sv, cv


def _sc_body(ts_hbm, tab_hbm, ao_hbm, bo_hbm, vo_hbm, po_hbm, ho_hbm,
             tab_v, ts_v, ao_v, bo_v, vo_v, po_v, ho_v, per_w):
    cid = lax.axis_index("c")
    sid = lax.axis_index("s")
    wid = sid * NC + cid
    base = wid * per_w
    pltpu.sync_copy(tab_hbm, tab_v)

    def chunk_body(ci, _):
        off = base + ci * CHUNK
        pltpu.sync_copy(ts_hbm.at[pl.ds(off, CHUNK)], ts_v)

        @plsc.parallel_loop(0, CHUNK, step=L, unroll=8)
        def vec_body(i):
            sl = pl.ds(i, L)
            ts = ts_v[sl]
            idx = jnp.zeros((L,), jnp.int32)
            for step in (16, 8, 4, 2, 1):
                tv = plsc.load_gather(tab_v, [idx + step])
                idx = jnp.where(tv <= ts, idx + step, idx)
            pv0 = plsc.load_gather(tab_v, [idx + 32])
            pacc = plsc.load_gather(tab_v, [idx + 64])
            pp0 = plsc.load_gather(tab_v, [idx + 96])
            pom = plsc.load_gather(tab_v, [idx + 128])
            ph = plsc.load_gather(tab_v, [idx + 160])
            pg = plsc.load_gather(tab_v, [idx + 192])
            pa0 = plsc.load_gather(tab_v, [idx + 224])
            pb0 = plsc.load_gather(tab_v, [idx + 256])
            phi_out = pp0 + pom * ts
            s, c = _sincos(phi_out)
            ao_v[sl] = pa0 + pg * s
            bo_v[sl] = pb0 - pg * c
            vo_v[sl] = pv0 + pacc * ts
            po_v[sl] = phi_out
            ho_v[sl] = ph
        dst = pl.ds(off, CHUNK)
        pltpu.sync_copy(ao_v, ao_hbm.at[dst])
        pltpu.sync_copy(bo_v, bo_hbm.at[dst])
        pltpu.sync_copy(vo_v, vo_hbm.at[dst])
        pltpu.sync_copy(po_v, po_hbm.at[dst])
        pltpu.sync_copy(ho_v, ho_hbm.at[dst])
        return 0

    lax.fori_loop(0, per_w // CHUNK, chunk_body, 0)


def kernel(timestamps, train_timestamp, a, b, v, phi, h):
    q = timestamps.shape[0]
    delta = jnp.diff(train_timestamp)
    acc = jnp.diff(v) / delta
    omega = jnp.diff(phi) / delta
    acc = jnp.concatenate([acc, acc[-1:]])
    omega = jnp.concatenate([omega, omega[-1:]])
    t = train_timestamp
    g = v / (omega + 1e-6)
    tab = jnp.concatenate(
        [t,
         v - acc * t,
         acc,
         phi - omega * t,
         omega,
         h,
         g,
         a - g * jnp.sin(phi),
         b + g * jnp.cos(phi),
         ]).astype(jnp.float32)

    grain = NW * CHUNK
    qp = ((q + grain - 1) // grain) * grain
    ts = timestamps
    if qp != q:
        ts = jnp.pad(ts, (0, qp - q))
    per_w = qp // NW

    mesh = plsc.VectorSubcoreMesh(core_axis_name="c", subcore_axis_name="s",
                                  num_cores=NC, num_subcores=NS)
    out = jax.ShapeDtypeStruct((qp,), jnp.float32)
    run = pl.kernel(
        functools.partial(_sc_body, per_w=per_w),
        out_type=(out, out, out, out, out),
        mesh=mesh,
        compiler_params=pltpu.CompilerParams(needs_layout_passes=False),
        scratch_types=[
            pltpu.VMEM((9 * NKEY,), jnp.float32),
            pltpu.VMEM((CHUNK,), jnp.float32),
            pltpu.VMEM((CHUNK,), jnp.float32),
            pltpu.VMEM((CHUNK,), jnp.float32),
            pltpu.VMEM((CHUNK,), jnp.float32),
            pltpu.VMEM((CHUNK,), jnp.float32),
            pltpu.VMEM((CHUNK,), jnp.float32),
        ],
    )
    a_out, b_out, v_out, phi_out, h_out = run(ts, tab)
    if qp != q:
        a_out, b_out, v_out, phi_out, h_out = (
            x[:q] for x in (a_out, b_out, v_out, phi_out, h_out))
    return (a_out, b_out, v_out, phi_out, h_out)


# speedup vs baseline: 10.9323x; 1.3735x over previous
import functools

import jax
import jax.numpy as jnp
from jax import lax
from jax.experimental import pallas as pl
from jax.experimental.pallas import tpu as pltpu
from jax.experimental.pallas import tpu_sc as plsc

NKEY = 32
NC = 2
NS = 16
L = 16
NW = NC * NS
CHUNK = 8192
UNROLL = 8


def _sincos(x):
    y = x * 0.6366197723675814 + 0.5
    k = y.astype(jnp.int32)
    k = k - jnp.where(y < 0.0, 1, 0)
    fn = k.astype(jnp.float32)
    r = x - fn * 1.5707962513
    r = r - fn * 7.54979013e-8
    r2 = r * r
    s = -1.9515295891e-4
    s = s * r2 + 8.3321608736e-3
    s = s * r2 - 1.6666654611e-1
    s = s * r2 * r + r
    c = 2.44331571e-5
    c = c * r2 - 1.38873163e-3
    c = c * r2 + 4.16666457e-2
    c = c * r2 * r2 - 0.5 * r2 + 1.0
    swap = (k & 1) != 0
    sv = jnp.where(swap, c, s)
    cv = jnp.where(swap, s, c)
    sv = jnp.where((k & 2) != 0, -sv, sv)
    cv = jnp.where(((k + 1) & 2) != 0, -cv, cv)
    return sv, cv


def _sc_body(ts_hbm, tab_hbm, ao, bo, vo, po, ho,
             tab_v, tsA, tsB,
             aoA, boA, voA, poA, hoA,
             aoB, boB, voB, poB, hoB,
             in_semA, in_semB, out_semA, out_semB, per_w):
    cid = lax.axis_index("c")
    sid = lax.axis_index("s")
    wid = sid * NC + cid
    base = wid * per_w
    n = per_w // CHUNK
    out_hbms = (ao, bo, vo, po, ho)
    bufsA = (aoA, boA, voA, poA, hoA)
    bufsB = (aoB, boB, voB, poB, hoB)

    pltpu.sync_copy(tab_hbm, tab_v)

    def in_dma(ci, buf, sem):
        return pltpu.make_async_copy(
            ts_hbm.at[pl.ds(base + ci * CHUNK, CHUNK)], buf, sem)

    def out_dmas(ci, bufs, sem):
        dst = pl.ds(base + ci * CHUNK, CHUNK)
        return [pltpu.make_async_copy(b, hbm.at[dst], sem)
                for b, hbm in zip(bufs, out_hbms)]

    def compute(ts_v, bufs):
        ao_v, bo_v, vo_v, po_v, ho_v = bufs

        @plsc.parallel_loop(0, CHUNK, step=L, unroll=UNROLL)
        def vec_body(i):
            sl = pl.ds(i, L)
            ts = ts_v[sl]
            idx = jnp.zeros((L,), jnp.int32)
            for step in (16, 8, 4, 2, 1):
                tv = plsc.load_gather(tab_v, [idx + step])
                idx = jnp.where(tv <= ts, idx + step, idx)
            pv0 = plsc.load_gather(tab_v, [idx + 32])
            pacc = plsc.load_gather(tab_v, [idx + 64])
            pp0 = plsc.load_gather(tab_v, [idx + 96])
            pom = plsc.load_gather(tab_v, [idx + 128])
            ph = plsc.load_gather(tab_v, [idx + 160])
            pg = plsc.load_gather(tab_v, [idx + 192])
            pa0 = plsc.load_gather(tab_v, [idx + 224])
            pb0 = plsc.load_gather(tab_v, [idx + 256])
            phi_out = pp0 + pom * ts
            s, c = _sincos(phi_out)
            ao_v[sl] = pa0 + pg * s
            bo_v[sl] = pb0 - pg * c
            vo_v[sl] = pv0 + pacc * ts
            po_v[sl] = phi_out
            ho_v[sl] = ph

    in_dma(0, tsA, in_semA).start()

    def pair_body(j, _):
        ci0 = 2 * j
        ci1 = 2 * j + 1
        in_dma(ci1, tsB, in_semB).start()
        in_dma(ci0, tsA, in_semA).wait()

        @pl.when(j > 0)
        def _():
            for d in out_dmas(ci0 - 2, bufsA, out_semA):
                d.wait()

        compute(tsA, bufsA)
        for d in out_dmas(ci0, bufsA, out_semA):
            d.start()

        @pl.when(j < (n // 2) - 1)
        def _():
            in_dma(ci1 + 1, tsA, in_semA).start()

        in_dma(ci1, tsB, in_semB).wait()

        @pl.when(j > 0)
        def _():
            for d in out_dmas(ci1 - 2, bufsB, out_semB):
                d.wait()

        compute(tsB, bufsB)
        for d in out_dmas(ci1, bufsB, out_semB):
            d.start()
        return 0

    lax.fori_loop(0, n // 2, pair_body, 0)
    for d in out_dmas(n - 2, bufsA, out_semA):
        d.wait()
    for d in out_dmas(n - 1, bufsB, out_semB):
        d.wait()


def kernel(timestamps, train_timestamp, a, b, v, phi, h):
    q = timestamps.shape[0]
    delta = jnp.diff(train_timestamp)
    acc = jnp.diff(v) / delta
    omega = jnp.diff(phi) / delta
    acc = jnp.concatenate([acc, acc[-1:]])
    omega = jnp.concatenate([omega, omega[-1:]])
    t = train_timestamp
    g = v / (omega + 1e-6)
    tab = jnp.concatenate(
        [t,
         v - acc * t,
         acc,
         phi - omega * t,
         omega,
         h,
         g,
         a - g * jnp.sin(phi),
         b + g * jnp.cos(phi),
         ]).astype(jnp.float32)

    grain = 2 * NW * CHUNK
    qp = ((q + grain - 1) // grain) * grain
    ts = timestamps
    if qp != q:
        ts = jnp.pad(ts, (0, qp - q))
    per_w = qp // NW

    mesh = plsc.VectorSubcoreMesh(core_axis_name="c", subcore_axis_name="s",
                                  num_cores=NC, num_subcores=NS)
    out = jax.ShapeDtypeStruct((qp,), jnp.float32)
    buf = pltpu.VMEM((CHUNK,), jnp.float32)
    run = pl.kernel(
        functools.partial(_sc_body, per_w=per_w),
        out_type=(out, out, out, out, out),
        mesh=mesh,
        compiler_params=pltpu.CompilerParams(needs_layout_passes=False),
        scratch_types=(
            [pltpu.VMEM((9 * NKEY,), jnp.float32)] + [buf] * 12
            + [pltpu.SemaphoreType.DMA] * 4
        ),
    )
    a_out, b_out, v_out, phi_out, h_out = run(ts, tab)
    if qp != q:
        a_out, b_out, v_out, phi_out, h_out = (
            x[:q] for x in (a_out, b_out, v_out, phi_out, h_out))
    return (a_out, b_out, v_out, phi_out, h_out)
